# Initial kernel scaffold; baseline (speedup 1.0000x reference)
import jax
import jax.numpy as jnp
from jax.experimental import pallas as pl


def kernel(x, feat, edge_index, edge_w, W_x, W_w, W_f, b_f, weights):
    raise NotImplementedError("write your pallas kernel here")


# speedup vs baseline: 3.2630x; 3.2630x over previous
import functools

import jax
import jax.numpy as jnp
from jax import lax
from jax.experimental import pallas as pl
from jax.experimental.pallas import tpu as pltpu
from jax.experimental.pallas import tpu_sc as plsc

N = 10000
D = 128
E = 320000

NC = 2
NS = 16
NW = NC * NS
K = 128
CPT = 80
EPT = CPT * K
E_PAD = NW * EPT
N_ACC = 10240
RPT = N_ACC // NS


def _sc_scatter(edata_hbm, ew_hbm, feat_hbm, zrow_hbm, zws_hbm,
                hf_out, ws_out,
                ebuf0, ebuf1, ewb0, ewb1, rows0, rows1, wv, di2,
                hf_sh, ws_sh, semg0, semg1, seme0, seme1):
    cid = lax.axis_index("c")
    sid = lax.axis_index("s")
    wid = cid * NS + sid

    pltpu.sync_copy(zrow_hbm, rows0)
    for k in range(RPT // K):
        pltpu.sync_copy(rows0, hf_sh.at[pl.ds(sid * RPT + k * K, K)])
    pltpu.sync_copy(zws_hbm, ws_sh.at[pl.ds(sid * 2 * RPT, 2 * RPT)])

    plsc.subcore_barrier()

    pltpu.async_copy(edata_hbm.at[wid, 0], ebuf0, seme0)
    pltpu.async_copy(ew_hbm.at[wid, 0], ewb0, seme0)

    def chunk(j, eb, ewb, rows_b, semg, eb_n, ewb_n, seme_n, seme_b):
        pltpu.make_async_copy(edata_hbm.at[wid, j], eb, seme_b).wait()
        pltpu.make_async_copy(ew_hbm.at[wid, j], ewb, seme_b).wait()
        cp = pltpu.async_copy(feat_hbm.at[eb.at[0]], rows_b, semg)
        pltpu.async_copy(edata_hbm.at[wid, j + 1], eb_n, seme_n)
        pltpu.async_copy(ew_hbm.at[wid, j + 1], ewb_n, seme_n)
        for v in range(K // 16):
            w = ewb[0, pl.ds(v * 16, 16)]
            d2 = eb[1, pl.ds(v * 16, 16)] * 2
            wv[0, pl.ds(v * 16, 16)] = jnp.maximum(w, 0.0)
            wv[1, pl.ds(v * 16, 16)] = jnp.maximum(-w, 0.0)
            di2[0, pl.ds(v * 16, 16)] = d2
            di2[1, pl.ds(v * 16, 16)] = d2 + 1
        pltpu.sync_copy(wv.at[0], ws_sh.at[di2.at[0]], add=True)
        pltpu.sync_copy(wv.at[1], ws_sh.at[di2.at[1]], add=True)
        cp.wait()
        pltpu.sync_copy(rows_b, hf_sh.at[eb.at[1]], add=True)

    def body(i, carry):
        chunk(2 * i, ebuf0, ewb0, rows0, semg0, ebuf1, ewb1, seme1, seme0)
        chunk(2 * i + 1, ebuf1, ewb1, rows1, semg1, ebuf0, ewb0, seme0, seme1)
        return carry

    lax.fori_loop(0, CPT // 2, body, 0)
    pltpu.make_async_copy(edata_hbm.at[wid, CPT], ebuf0, seme0).wait()
    pltpu.make_async_copy(ew_hbm.at[wid, CPT], ewb0, seme0).wait()
    plsc.subcore_barrier()

    pltpu.sync_copy(hf_sh.at[pl.ds(sid * RPT, RPT)],
                    hf_out.at[cid, pl.ds(sid * RPT, RPT)])
    pltpu.sync_copy(ws_sh.at[pl.ds(sid * 2 * RPT, 2 * RPT)],
                    ws_out.at[cid, pl.ds(sid * 2 * RPT, 2 * RPT)])


def _sc_call(edata, ew4, feat, zrow, zws):
    mesh = plsc.VectorSubcoreMesh(core_axis_name="c", subcore_axis_name="s")
    f = pl.kernel(
        _sc_scatter,
        out_type=[
            jax.ShapeDtypeStruct((NC, N_ACC, D), jnp.float32),
            jax.ShapeDtypeStruct((NC, 2 * N_ACC), jnp.float32),
        ],
        mesh=mesh,
        scratch_types=[
            pltpu.VMEM((2, K), jnp.int32),
            pltpu.VMEM((2, K), jnp.int32),
            pltpu.VMEM((1, K), jnp.float32),
            pltpu.VMEM((1, K), jnp.float32),
            pltpu.VMEM((K, D), jnp.float32),
            pltpu.VMEM((K, D), jnp.float32),
            pltpu.VMEM((2, K), jnp.float32),
            pltpu.VMEM((2, K), jnp.int32),
            pltpu.VMEM_SHARED((N_ACC, D), jnp.float32),
            pltpu.VMEM_SHARED((2 * N_ACC,), jnp.float32),
            pltpu.SemaphoreType.DMA,
            pltpu.SemaphoreType.DMA,
            pltpu.SemaphoreType.DMA,
            pltpu.SemaphoreType.DMA,
        ],
    )
    return f(edata, ew4, feat, zrow, zws)


def _tc_epilogue(x_ref, hf_ref, ws_ref, wx_ref, wf_ref, ww_ref, b_ref, wt_ref,
                 out_ref):
    f32 = jnp.float32
    wt = wt_ref[...]
    rw = jnp.concatenate([jnp.maximum(wt, 0.0), jnp.maximum(-wt, 0.0)], axis=0)
    v = lax.dot_general(rw, ww_ref[...], (((1,), (1,)), ((), ())),
                        preferred_element_type=f32)
    s = ws_ref[0] + ws_ref[1]
    hf = hf_ref[0] + hf_ref[1]
    acc = lax.dot_general(x_ref[...], wx_ref[...], (((1,), (1,)), ((), ())),
                          preferred_element_type=f32)
    acc += lax.dot_general(hf, wf_ref[...], (((1,), (1,)), ((), ())),
                           preferred_element_type=f32)
    acc += lax.dot_general(s, v, (((1,), (0,)), ((), ())),
                           preferred_element_type=f32)
    acc += b_ref[...]
    out_ref[...] = jnp.maximum(acc, 0.0)


def _tc_call(x, hf, ws, W_x, W_f, W_w, b_f, weights):
    blk = 1000
    grid = (N // blk,)
    return pl.pallas_call(
        _tc_epilogue,
        grid=grid,
        in_specs=[
            pl.BlockSpec((blk, D), lambda i: (i, 0)),
            pl.BlockSpec((NC, blk, D), lambda i: (0, i, 0)),
            pl.BlockSpec((NC, blk, 2), lambda i: (0, i, 0)),
            pl.BlockSpec((D, D), lambda i: (0, 0)),
            pl.BlockSpec((D, D), lambda i: (0, 0)),
            pl.BlockSpec((D, D), lambda i: (0, 0)),
            pl.BlockSpec((1, D), lambda i: (0, 0)),
            pl.BlockSpec((1, D), lambda i: (0, 0)),
        ],
        out_specs=pl.BlockSpec((blk, D), lambda i: (i, 0)),
        out_shape=jax.ShapeDtypeStruct((N, D), jnp.float32),
    )(x, hf, ws, W_x, W_f, W_w, b_f, weights)


@jax.jit
def kernel(x, feat, edge_index, edge_w, W_x, W_w, W_f, b_f, weights):
    src = edge_index[0].astype(jnp.int32)
    dst = edge_index[1].astype(jnp.int32)
    pad = E_PAD - E
    src3 = jnp.concatenate([src, jnp.zeros((pad,), jnp.int32)]).reshape(NW, CPT, K)
    dst3 = jnp.concatenate([dst, jnp.full((pad,), N, jnp.int32)]).reshape(NW, CPT, K)
    ew3 = jnp.concatenate([edge_w, jnp.zeros((pad,), jnp.float32)]).reshape(NW, CPT, K)
    edata = jnp.stack([src3, dst3], axis=2)
    dummy = jnp.stack([jnp.zeros((NW, 1, K), jnp.int32),
                       jnp.full((NW, 1, K), N, jnp.int32)], axis=2)
    edata = jnp.concatenate([edata, dummy], axis=1)
    ew4 = jnp.concatenate([ew3[:, :, None, :],
                           jnp.zeros((NW, 1, 1, K), jnp.float32)], axis=1)
    zrow = jnp.zeros((K, D), jnp.float32)
    zws = jnp.zeros((2 * RPT,), jnp.float32)
    hf, ws = _sc_call(edata, ew4, feat, zrow, zws)
    ws = ws.reshape(NC, N_ACC, 2)
    return _tc_call(x, hf, ws, W_x, W_f, W_w, b_f.reshape(1, D),
                    weights.reshape(1, D))


# speedup vs baseline: 3.7235x; 1.1411x over previous
import functools

import jax
import jax.numpy as jnp
from jax import lax
from jax.experimental import pallas as pl
from jax.experimental.pallas import tpu as pltpu
from jax.experimental.pallas import tpu_sc as plsc

N = 10000
D = 128
E = 320000

NC = 2
NS = 16
NW = NC * NS
K = 128
CPT = 80
EPT = CPT * K
E_PAD = NW * EPT
N_ACC = 10240
RPT = N_ACC // NS


def _sc_scatter(edata_hbm, ew_hbm, feat_hbm, zrow_hbm, zws_hbm,
                hf_out, ws_out,
                ebuf0, ebuf1, ewb0, ewb1, rows0, rows1, wv, di2,
                hf_sh, ws_sh, semg0, semg1, seme0, seme1):
    cid = lax.axis_index("c")
    sid = lax.axis_index("s")
    wid = cid * NS + sid

    pltpu.sync_copy(zrow_hbm, rows0)
    for k in range(RPT // K):
        pltpu.sync_copy(rows0, hf_sh.at[pl.ds(sid * RPT + k * K, K)])
    pltpu.sync_copy(zws_hbm, ws_sh.at[pl.ds(sid * 2 * RPT, 2 * RPT)])

    plsc.subcore_barrier()

    pltpu.async_copy(edata_hbm.at[wid, 0], ebuf0, seme0)
    pltpu.async_copy(ew_hbm.at[wid, 0], ewb0, seme0)

    def chunk(j, eb, ewb, rows_b, semg, eb_n, ewb_n, seme_n, seme_b):
        pltpu.make_async_copy(edata_hbm.at[wid, j], eb, seme_b).wait()
        pltpu.make_async_copy(ew_hbm.at[wid, j], ewb, seme_b).wait()
        cp = pltpu.async_copy(feat_hbm.at[eb.at[0]], rows_b, semg)
        pltpu.async_copy(edata_hbm.at[wid, j + 1], eb_n, seme_n)
        pltpu.async_copy(ew_hbm.at[wid, j + 1], ewb_n, seme_n)
        for v in range(K // 16):
            w = ewb[0, pl.ds(v * 16, 16)]
            d2 = eb[1, pl.ds(v * 16, 16)] * 2
            wv[0, pl.ds(v * 16, 16)] = jnp.maximum(w, 0.0)
            wv[1, pl.ds(v * 16, 16)] = jnp.maximum(-w, 0.0)
            di2[0, pl.ds(v * 16, 16)] = d2
            di2[1, pl.ds(v * 16, 16)] = d2 + 1
        pltpu.sync_copy(wv.at[0], ws_sh.at[di2.at[0]], add=True)
        pltpu.sync_copy(wv.at[1], ws_sh.at[di2.at[1]], add=True)
        cp.wait()
        pltpu.sync_copy(rows_b, hf_sh.at[eb.at[1]], add=True)

    def body(i, carry):
        chunk(2 * i, ebuf0, ewb0, rows0, semg0, ebuf1, ewb1, seme1, seme0)
        chunk(2 * i + 1, ebuf1, ewb1, rows1, semg1, ebuf0, ewb0, seme0, seme1)
        return carry

    lax.fori_loop(0, CPT // 2, body, 0)
    pltpu.make_async_copy(edata_hbm.at[wid, CPT], ebuf0, seme0).wait()
    pltpu.make_async_copy(ew_hbm.at[wid, CPT], ewb0, seme0).wait()
    plsc.subcore_barrier()

    pltpu.sync_copy(hf_sh.at[pl.ds(sid * RPT, RPT)],
                    hf_out.at[cid, pl.ds(sid * RPT, RPT)])
    pltpu.sync_copy(ws_sh.at[pl.ds(sid * 2 * RPT, 2 * RPT)],
                    ws_out.at[cid, pl.ds(sid * 2 * RPT, 2 * RPT)])


def _sc_call(edata, ew4, feat, zrow, zws):
    mesh = plsc.VectorSubcoreMesh(core_axis_name="c", subcore_axis_name="s")
    f = pl.kernel(
        _sc_scatter,
        out_type=[
            jax.ShapeDtypeStruct((NC, N_ACC, D), jnp.float32),
            jax.ShapeDtypeStruct((NC, 2 * N_ACC), jnp.float32),
        ],
        mesh=mesh,
        scratch_types=[
            pltpu.VMEM((2, K), jnp.int32),
            pltpu.VMEM((2, K), jnp.int32),
            pltpu.VMEM((1, K), jnp.float32),
            pltpu.VMEM((1, K), jnp.float32),
            pltpu.VMEM((K, D), jnp.float32),
            pltpu.VMEM((K, D), jnp.float32),
            pltpu.VMEM((2, K), jnp.float32),
            pltpu.VMEM((2, K), jnp.int32),
            pltpu.VMEM_SHARED((N_ACC, D), jnp.float32),
            pltpu.VMEM_SHARED((2 * N_ACC,), jnp.float32),
            pltpu.SemaphoreType.DMA,
            pltpu.SemaphoreType.DMA,
            pltpu.SemaphoreType.DMA,
            pltpu.SemaphoreType.DMA,
        ],
    )
    return f(edata, ew4, feat, zrow, zws)


def _tc_epilogue(x_ref, hf_ref, ws_ref, wx_ref, wf_ref, ww_ref, b_ref, wt_ref,
                 out_ref):
    f32 = jnp.float32
    wt = wt_ref[...]
    rw = jnp.concatenate([jnp.maximum(wt, 0.0), jnp.maximum(-wt, 0.0)], axis=0)
    v = lax.dot_general(rw, ww_ref[...], (((1,), (1,)), ((), ())),
                        preferred_element_type=f32)
    s = ws_ref[0] + ws_ref[1]
    hf = hf_ref[0] + hf_ref[1]
    acc = lax.dot_general(x_ref[...], wx_ref[...], (((1,), (1,)), ((), ())),
                          preferred_element_type=f32)
    acc += lax.dot_general(hf, wf_ref[...], (((1,), (1,)), ((), ())),
                           preferred_element_type=f32)
    acc += lax.dot_general(s, v, (((1,), (0,)), ((), ())),
                           preferred_element_type=f32)
    acc += b_ref[...]
    out_ref[...] = jnp.maximum(acc, 0.0)


def _tc_call(x, hf, ws, W_x, W_f, W_w, b_f, weights):
    blk = 1000
    grid = (N // blk,)
    return pl.pallas_call(
        _tc_epilogue,
        grid=grid,
        in_specs=[
            pl.BlockSpec((blk, D), lambda i: (i, 0)),
            pl.BlockSpec((NC, blk, D), lambda i: (0, i, 0)),
            pl.BlockSpec((NC, blk, 2), lambda i: (0, i, 0)),
            pl.BlockSpec((D, D), lambda i: (0, 0)),
            pl.BlockSpec((D, D), lambda i: (0, 0)),
            pl.BlockSpec((D, D), lambda i: (0, 0)),
            pl.BlockSpec((1, D), lambda i: (0, 0)),
            pl.BlockSpec((1, D), lambda i: (0, 0)),
        ],
        out_specs=pl.BlockSpec((blk, D), lambda i: (i, 0)),
        out_shape=jax.ShapeDtypeStruct((N, D), jnp.float32),
    )(x, hf, ws, W_x, W_f, W_w, b_f, weights)


@jax.jit
def kernel(x, feat, edge_index, edge_w, W_x, W_w, W_f, b_f, weights):
    src = edge_index[0].astype(jnp.int32)
    dst = edge_index[1].astype(jnp.int32)
    pad = E_PAD - E
    pad_dst = N + jnp.arange(pad, dtype=jnp.int32) % (N_ACC - N)
    src3 = jnp.concatenate([src, jnp.zeros((pad,), jnp.int32)]).reshape(NW, CPT, K)
    dst3 = jnp.concatenate([dst, pad_dst]).reshape(NW, CPT, K)
    ew3 = jnp.concatenate([edge_w, jnp.zeros((pad,), jnp.float32)]).reshape(NW, CPT, K)
    edata = jnp.stack([src3, dst3], axis=2)
    dummy = jnp.stack([jnp.zeros((NW, 1, K), jnp.int32),
                       jnp.full((NW, 1, K), N, jnp.int32)], axis=2)
    edata = jnp.concatenate([edata, dummy], axis=1)
    ew4 = jnp.concatenate([ew3[:, :, None, :],
                           jnp.zeros((NW, 1, 1, K), jnp.float32)], axis=1)
    zrow = jnp.zeros((K, D), jnp.float32)
    zws = jnp.zeros((2 * RPT,), jnp.float32)
    hf, ws = _sc_call(edata, ew4, feat, zrow, zws)
    ws = ws.reshape(NC, N_ACC, 2)
    return _tc_call(x, hf, ws, W_x, W_f, W_w, b_f.reshape(1, D),
                    weights.reshape(1, D))


# speedup vs baseline: 4.3569x; 1.1701x over previous
import functools

import jax
import jax.numpy as jnp
from jax import lax
from jax.experimental import pallas as pl
from jax.experimental.pallas import tpu as pltpu
from jax.experimental.pallas import tpu_sc as plsc

N = 10000
D = 128
E = 320000

NC = 2
NS = 16
NW = NC * NS
K = 128
CPT0 = 120
CPT1 = 40
NCH = NS * (CPT0 + CPT1)
E_PAD = NCH * K
N_ACC = 10240
RPT = N_ACC // NS


def _sc_scatter(edata_hbm, ew_hbm, feat_hbm, zrow_hbm, zws_hbm,
                hf_out, ws_out,
                ebuf0, ebuf1, ewb0, ewb1, rows0, rows1, wv, di2,
                hf_sh, ws_sh, semg0, semg1, seme0, seme1):
    cid = lax.axis_index("c")
    sid = lax.axis_index("s")
    base = jnp.where(cid == 0, sid * CPT0, NS * CPT0 + sid * CPT1)
    half_chunks = jnp.where(cid == 0, CPT0 // 2, CPT1 // 2)

    pltpu.sync_copy(zrow_hbm, rows0)
    for k in range(RPT // K):
        pltpu.sync_copy(rows0, hf_sh.at[pl.ds(sid * RPT + k * K, K)])
    pltpu.sync_copy(zws_hbm, ws_sh.at[pl.ds(sid * 2 * RPT, 2 * RPT)])

    plsc.subcore_barrier()

    pltpu.async_copy(edata_hbm.at[base], ebuf0, seme0)
    pltpu.async_copy(ew_hbm.at[base], ewb0, seme0)

    def chunk(j, eb, ewb, rows_b, semg, eb_n, ewb_n, seme_n, seme_b):
        pltpu.make_async_copy(edata_hbm.at[base + j], eb, seme_b).wait()
        pltpu.make_async_copy(ew_hbm.at[base + j], ewb, seme_b).wait()
        cp = pltpu.async_copy(feat_hbm.at[eb.at[0]], rows_b, semg)
        pltpu.async_copy(edata_hbm.at[base + j + 1], eb_n, seme_n)
        pltpu.async_copy(ew_hbm.at[base + j + 1], ewb_n, seme_n)
        for v in range(K // 16):
            w = ewb[0, pl.ds(v * 16, 16)]
            d2 = eb[1, pl.ds(v * 16, 16)] * 2
            wv[0, pl.ds(v * 16, 16)] = jnp.maximum(w, 0.0)
            wv[1, pl.ds(v * 16, 16)] = jnp.maximum(-w, 0.0)
            di2[0, pl.ds(v * 16, 16)] = d2
            di2[1, pl.ds(v * 16, 16)] = d2 + 1
        pltpu.sync_copy(wv.at[0], ws_sh.at[di2.at[0]], add=True)
        pltpu.sync_copy(wv.at[1], ws_sh.at[di2.at[1]], add=True)
        cp.wait()
        pltpu.sync_copy(rows_b, hf_sh.at[eb.at[1]], add=True)

    def body(i, carry):
        chunk(2 * i, ebuf0, ewb0, rows0, semg0, ebuf1, ewb1, seme1, seme0)
        chunk(2 * i + 1, ebuf1, ewb1, rows1, semg1, ebuf0, ewb0, seme0, seme1)
        return carry

    lax.fori_loop(0, half_chunks, body, 0)
    pltpu.make_async_copy(edata_hbm.at[base], ebuf0, seme0).wait()
    pltpu.make_async_copy(ew_hbm.at[base], ewb0, seme0).wait()
    plsc.subcore_barrier()

    pltpu.sync_copy(hf_sh.at[pl.ds(sid * RPT, RPT)],
                    hf_out.at[cid, pl.ds(sid * RPT, RPT)])
    pltpu.sync_copy(ws_sh.at[pl.ds(sid * 2 * RPT, 2 * RPT)],
                    ws_out.at[cid, pl.ds(sid * 2 * RPT, 2 * RPT)])


def _sc_call(edata, ew4, feat, zrow, zws):
    mesh = plsc.VectorSubcoreMesh(core_axis_name="c", subcore_axis_name="s")
    f = pl.kernel(
        _sc_scatter,
        out_type=[
            jax.ShapeDtypeStruct((NC, N_ACC, D), jnp.float32),
            jax.ShapeDtypeStruct((NC, 2 * N_ACC), jnp.float32),
        ],
        mesh=mesh,
        scratch_types=[
            pltpu.VMEM((2, K), jnp.int32),
            pltpu.VMEM((2, K), jnp.int32),
            pltpu.VMEM((1, K), jnp.float32),
            pltpu.VMEM((1, K), jnp.float32),
            pltpu.VMEM((K, D), jnp.float32),
            pltpu.VMEM((K, D), jnp.float32),
            pltpu.VMEM((2, K), jnp.float32),
            pltpu.VMEM((2, K), jnp.int32),
            pltpu.VMEM_SHARED((N_ACC, D), jnp.float32),
            pltpu.VMEM_SHARED((2 * N_ACC,), jnp.float32),
            pltpu.SemaphoreType.DMA,
            pltpu.SemaphoreType.DMA,
            pltpu.SemaphoreType.DMA,
            pltpu.SemaphoreType.DMA,
        ],
    )
    return f(edata, ew4, feat, zrow, zws)


def _tc_epilogue(x_ref, hf_ref, ws_ref, wx_ref, wf_ref, ww_ref, b_ref, wt_ref,
                 out_ref):
    f32 = jnp.float32
    wt = wt_ref[...]
    rw = jnp.concatenate([jnp.maximum(wt, 0.0), jnp.maximum(-wt, 0.0)], axis=0)
    v = lax.dot_general(rw, ww_ref[...], (((1,), (1,)), ((), ())),
                        preferred_element_type=f32)
    s = ws_ref[0] + ws_ref[1]
    hf = hf_ref[0] + hf_ref[1]
    acc = lax.dot_general(x_ref[...], wx_ref[...], (((1,), (1,)), ((), ())),
                          preferred_element_type=f32)
    acc += lax.dot_general(hf, wf_ref[...], (((1,), (1,)), ((), ())),
                           preferred_element_type=f32)
    acc += lax.dot_general(s, v, (((1,), (0,)), ((), ())),
                           preferred_element_type=f32)
    acc += b_ref[...]
    out_ref[...] = jnp.maximum(acc, 0.0)


def _tc_call(x, hf, ws, W_x, W_f, W_w, b_f, weights):
    blk = 1000
    grid = (N // blk,)
    return pl.pallas_call(
        _tc_epilogue,
        grid=grid,
        in_specs=[
            pl.BlockSpec((blk, D), lambda i: (i, 0)),
            pl.BlockSpec((NC, blk, D), lambda i: (0, i, 0)),
            pl.BlockSpec((NC, blk, 2), lambda i: (0, i, 0)),
            pl.BlockSpec((D, D), lambda i: (0, 0)),
            pl.BlockSpec((D, D), lambda i: (0, 0)),
            pl.BlockSpec((D, D), lambda i: (0, 0)),
            pl.BlockSpec((1, D), lambda i: (0, 0)),
            pl.BlockSpec((1, D), lambda i: (0, 0)),
        ],
        out_specs=pl.BlockSpec((blk, D), lambda i: (i, 0)),
        out_shape=jax.ShapeDtypeStruct((N, D), jnp.float32),
    )(x, hf, ws, W_x, W_f, W_w, b_f, weights)


@jax.jit
def kernel(x, feat, edge_index, edge_w, W_x, W_w, W_f, b_f, weights):
    src = edge_index[0].astype(jnp.int32)
    dst = edge_index[1].astype(jnp.int32)
    pad = E_PAD - E
    pad_dst = N + jnp.arange(pad, dtype=jnp.int32) % (N_ACC - N)
    src2 = jnp.concatenate([src, jnp.zeros((pad,), jnp.int32)]).reshape(NCH, K)
    dst2 = jnp.concatenate([dst, pad_dst]).reshape(NCH, K)
    ew2 = jnp.concatenate([edge_w, jnp.zeros((pad,), jnp.float32)]).reshape(NCH, K)
    edata = jnp.stack([src2, dst2], axis=1)
    dummy = jnp.stack([jnp.zeros((1, K), jnp.int32),
                       jnp.full((1, K), N, jnp.int32)], axis=1)
    edata = jnp.concatenate([edata, dummy], axis=0)
    ew4 = jnp.concatenate([ew2, jnp.zeros((1, K), jnp.float32)],
                          axis=0).reshape(NCH + 1, 1, K)
    zrow = jnp.zeros((K, D), jnp.float32)
    zws = jnp.zeros((2 * RPT,), jnp.float32)
    hf, ws = _sc_call(edata, ew4, feat, zrow, zws)
    ws = ws.reshape(NC, N_ACC, 2)
    return _tc_call(x, hf, ws, W_x, W_f, W_w, b_f.reshape(1, D),
                    weights.reshape(1, D))
